# baseline (speedup 1.0000x reference)
import functools

import jax
import jax.numpy as jnp
from jax import lax
from jax.experimental import pallas as pl
from jax.experimental.pallas import tpu as pltpu
from jax.experimental.pallas import tpu_sc as plsc

B = 16384
D = 64
K = 3
NC = 2
NS = 16
NW = NC * NS
PER_W = B // NW
CHUNK = 128
NCHUNK = PER_W // CHUNK


def _sc_scores_body(embT, cen, ctx, ng0, ng1, ng2, out,
                    cidx, xidx, i0, i1, i2,
                    ub, vb, b0, b1, b2,
                    pbuf, nb0, nb1, nb2, sem):
    wid = lax.axis_index("s") * NC + lax.axis_index("c")
    for c in range(NCHUNK):
        base = wid * PER_W + c * CHUNK
        pltpu.sync_copy(cen.at[pl.ds(base, CHUNK)], cidx)
        pltpu.sync_copy(ctx.at[pl.ds(base, CHUNK)], xidx)
        pltpu.sync_copy(ng0.at[pl.ds(base, CHUNK)], i0)
        pltpu.sync_copy(ng1.at[pl.ds(base, CHUNK)], i1)
        pltpu.sync_copy(ng2.at[pl.ds(base, CHUNK)], i2)

        def dfire(d, carry):
            row = embT.at[d]
            pltpu.async_copy(row.at[cidx], ub.at[d], sem)
            pltpu.async_copy(row.at[xidx], vb.at[d], sem)
            pltpu.async_copy(row.at[i0], b0.at[d], sem)
            pltpu.async_copy(row.at[i1], b1.at[d], sem)
            pltpu.async_copy(row.at[i2], b2.at[d], sem)
            return carry

        lax.fori_loop(0, D, dfire, 0)
        dummy = embT.at[pl.ds(0, D), pl.ds(0, CHUNK)]
        pltpu.make_async_copy(dummy, ub, sem).wait()
        pltpu.make_async_copy(dummy, vb, sem).wait()
        pltpu.make_async_copy(dummy, b0, sem).wait()
        pltpu.make_async_copy(dummy, b1, sem).wait()
        pltpu.make_async_copy(dummy, b2, sem).wait()

        zero = jnp.zeros((16,), jnp.float32)

        def gbody(g, carry):
            o = g * 16
            pa, a0, a1, a2 = zero, zero, zero, zero
            for d in range(D):
                u = ub[d, pl.ds(o, 16)]
                pa = pa + u * vb[d, pl.ds(o, 16)]
                a0 = a0 + u * b0[d, pl.ds(o, 16)]
                a1 = a1 + u * b1[d, pl.ds(o, 16)]
                a2 = a2 + u * b2[d, pl.ds(o, 16)]
            pbuf[pl.ds(o, 16)] = pa
            nb0[pl.ds(o, 16)] = a0
            nb1[pl.ds(o, 16)] = a1
            nb2[pl.ds(o, 16)] = a2
            return carry

        lax.fori_loop(0, CHUNK // 16, gbody, 0)
        pltpu.sync_copy(pbuf, out.at[0, pl.ds(base, CHUNK)])
        pltpu.sync_copy(nb0, out.at[1, pl.ds(base, CHUNK)])
        pltpu.sync_copy(nb1, out.at[2, pl.ds(base, CHUNK)])
        pltpu.sync_copy(nb2, out.at[3, pl.ds(base, CHUNK)])


_sc_scores = functools.partial(
    pl.kernel,
    out_type=jax.ShapeDtypeStruct((K + 1, B), jnp.float32),
    mesh=plsc.VectorSubcoreMesh(
        core_axis_name="c", subcore_axis_name="s",
        num_cores=NC, num_subcores=NS),
    compiler_params=pltpu.CompilerParams(
        needs_layout_passes=False, use_tc_tiling_on_sc=False),
    scratch_types=[
        pltpu.VMEM((CHUNK,), jnp.int32),
        pltpu.VMEM((CHUNK,), jnp.int32),
        pltpu.VMEM((CHUNK,), jnp.int32),
        pltpu.VMEM((CHUNK,), jnp.int32),
        pltpu.VMEM((CHUNK,), jnp.int32),
        pltpu.VMEM((D, CHUNK), jnp.float32),
        pltpu.VMEM((D, CHUNK), jnp.float32),
        pltpu.VMEM((D, CHUNK), jnp.float32),
        pltpu.VMEM((D, CHUNK), jnp.float32),
        pltpu.VMEM((D, CHUNK), jnp.float32),
        pltpu.VMEM((CHUNK,), jnp.float32),
        pltpu.VMEM((CHUNK,), jnp.float32),
        pltpu.VMEM((CHUNK,), jnp.float32),
        pltpu.VMEM((CHUNK,), jnp.float32),
        pltpu.SemaphoreType.DMA,
    ],
)(_sc_scores_body)


def _loss_body(s_ref, o_ref):
    x = s_ref[...]
    row = lax.broadcasted_iota(jnp.int32, x.shape, 0)
    ispos = row == 0
    s = jnp.where(ispos, x, -x)
    ls = jnp.minimum(s, 0.0) - jnp.log1p(jnp.exp(-jnp.abs(s)))
    pos_sum = jnp.sum(jnp.where(ispos, ls, 0.0))
    neg_sum = jnp.sum(jnp.where(ispos, 0.0, ls))
    o_ref[0, 0] = -(pos_sum / B) - (neg_sum / (K * B))


_loss = pl.pallas_call(
    _loss_body,
    out_shape=jax.ShapeDtypeStruct((1, 1), jnp.float32),
    out_specs=pl.BlockSpec(memory_space=pltpu.SMEM),
)


@jax.jit
def _impl(centers, contexts, neg_samples, emb):
    cen = centers.astype(jnp.int32)
    ctx = contexts.astype(jnp.int32)
    neg = neg_samples.astype(jnp.int32)
    embT = jnp.transpose(emb)
    scores = _sc_scores(embT, cen, ctx, neg[:, 0], neg[:, 1], neg[:, 2])
    return _loss(scores)[0, 0]


def kernel(centers, contexts, neg_samples, emb):
    return _impl(centers, contexts, neg_samples, emb)


# speedup vs baseline: 2.2227x; 2.2227x over previous
import functools

import jax
import jax.numpy as jnp
from jax import lax
from jax.experimental import pallas as pl
from jax.experimental.pallas import tpu as pltpu
from jax.experimental.pallas import tpu_sc as plsc

B = 16384
D = 64
K = 3
NC = 2
NS = 16
NW = NC * NS
PER_W = B // NW
CHUNK = 128
NCHUNK = PER_W // CHUNK
W = 2 * D
TROWS = 500000
NBLK = 3907


def _tp_body(a_ref, b_ref, o_ref):
    o_ref[...] = jnp.concatenate(
        [a_ref[...].T, b_ref[...].T], axis=1)


_transpose = pl.pallas_call(
    _tp_body,
    grid=(NBLK,),
    in_specs=[
        pl.BlockSpec((D, 128), lambda j: (0, 2 * j)),
        pl.BlockSpec((D, 128), lambda j: (0, 2 * j + 1)),
    ],
    out_specs=pl.BlockSpec((128, W), lambda j: (j, 0)),
    out_shape=jax.ShapeDtypeStruct((TROWS, W), jnp.float32),
)


def _sc_scores_body(emb, cen_h, ctx_h, neg_h, cen_o, ctx_o, neg_o, out,
                    cidx, xidx, nidx, coff, xoff, noff,
                    urows, vrows, nrows, pbuf, nb0, nb1, nb2, sem):
    wid = lax.axis_index("s") * NC + lax.axis_index("c")
    lanes = lax.iota(jnp.int32, 16)
    for c in range(NCHUNK):
        base = wid * PER_W + c * CHUNK
        pltpu.sync_copy(cen_h.at[pl.ds(base, CHUNK)], cidx)
        pltpu.sync_copy(ctx_h.at[pl.ds(base, CHUNK)], xidx)
        pltpu.sync_copy(neg_h.at[pl.ds(base * K, CHUNK * K)], nidx)
        pltpu.sync_copy(cen_o.at[pl.ds(base, CHUNK)], coff)
        pltpu.sync_copy(ctx_o.at[pl.ds(base, CHUNK)], xoff)
        pltpu.sync_copy(neg_o.at[pl.ds(base * K, CHUNK * K)], noff)
        d1 = pltpu.async_copy(emb.at[cidx], urows, sem)
        d2 = pltpu.async_copy(emb.at[xidx], vrows, sem)
        d3 = pltpu.async_copy(emb.at[nidx.at[pl.ds(0, 128)]],
                              nrows.at[pl.ds(0, 128)], sem)
        d4 = pltpu.async_copy(emb.at[nidx.at[pl.ds(128, 128)]],
                              nrows.at[pl.ds(128, 128)], sem)
        d5 = pltpu.async_copy(emb.at[nidx.at[pl.ds(256, 128)]],
                              nrows.at[pl.ds(256, 128)], sem)
        d1.wait(); d2.wait(); d3.wait(); d4.wait(); d5.wait()

        zero = jnp.zeros((16,), jnp.float32)

        def gbody(g, carry):
            rows = lanes + g * 16
            rows3 = rows * K
            uoff = coff[pl.ds(g * 16, 16)]
            voff = xoff[pl.ds(g * 16, 16)]
            w0 = plsc.load_gather(noff, [rows3])
            w1 = plsc.load_gather(noff, [rows3 + 1])
            w2 = plsc.load_gather(noff, [rows3 + 2])
            pa, a0, a1, a2 = zero, zero, zero, zero
            for d in range(D):
                u = plsc.load_gather(urows, [rows, uoff + d])
                v = plsc.load_gather(vrows, [rows, voff + d])
                m0 = plsc.load_gather(nrows, [rows3, w0 + d])
                m1 = plsc.load_gather(nrows, [rows3 + 1, w1 + d])
                m2 = plsc.load_gather(nrows, [rows3 + 2, w2 + d])
                pa = pa + u * v
                a0 = a0 + u * m0
                a1 = a1 + u * m1
                a2 = a2 + u * m2
            pbuf[pl.ds(g * 16, 16)] = pa
            nb0[pl.ds(g * 16, 16)] = a0
            nb1[pl.ds(g * 16, 16)] = a1
            nb2[pl.ds(g * 16, 16)] = a2
            return carry

        lax.fori_loop(0, CHUNK // 16, gbody, 0)
        pltpu.sync_copy(pbuf, out.at[0, pl.ds(base, CHUNK)])
        pltpu.sync_copy(nb0, out.at[1, pl.ds(base, CHUNK)])
        pltpu.sync_copy(nb1, out.at[2, pl.ds(base, CHUNK)])
        pltpu.sync_copy(nb2, out.at[3, pl.ds(base, CHUNK)])


_sc_scores = functools.partial(
    pl.kernel,
    out_type=jax.ShapeDtypeStruct((K + 1, B), jnp.float32),
    mesh=plsc.VectorSubcoreMesh(
        core_axis_name="c", subcore_axis_name="s",
        num_cores=NC, num_subcores=NS),
    compiler_params=pltpu.CompilerParams(needs_layout_passes=False),
    scratch_types=[
        pltpu.VMEM((CHUNK,), jnp.int32),
        pltpu.VMEM((CHUNK,), jnp.int32),
        pltpu.VMEM((CHUNK * K,), jnp.int32),
        pltpu.VMEM((CHUNK,), jnp.int32),
        pltpu.VMEM((CHUNK,), jnp.int32),
        pltpu.VMEM((CHUNK * K,), jnp.int32),
        pltpu.VMEM((CHUNK, W), jnp.float32),
        pltpu.VMEM((CHUNK, W), jnp.float32),
        pltpu.VMEM((CHUNK * K, W), jnp.float32),
        pltpu.VMEM((CHUNK,), jnp.float32),
        pltpu.VMEM((CHUNK,), jnp.float32),
        pltpu.VMEM((CHUNK,), jnp.float32),
        pltpu.VMEM((CHUNK,), jnp.float32),
        pltpu.SemaphoreType.DMA,
    ],
)(_sc_scores_body)


def _loss_body(s_ref, o_ref):
    x = s_ref[...]
    row = lax.broadcasted_iota(jnp.int32, x.shape, 0)
    ispos = row == 0
    s = jnp.where(ispos, x, -x)
    ls = jnp.minimum(s, 0.0) - jnp.log1p(jnp.exp(-jnp.abs(s)))
    pos_sum = jnp.sum(jnp.where(ispos, ls, 0.0))
    neg_sum = jnp.sum(jnp.where(ispos, 0.0, ls))
    o_ref[0, 0] = -(pos_sum / B) - (neg_sum / (K * B))


_loss = pl.pallas_call(
    _loss_body,
    out_shape=jax.ShapeDtypeStruct((1, 1), jnp.float32),
    out_specs=pl.BlockSpec(memory_space=pltpu.SMEM),
)


def _pack_row(i):
    return ((i >> 8) << 7) + (i & 127)


def _pack_off(i):
    return (i & 128) >> 1


@jax.jit
def _impl(centers, contexts, neg_samples, emb):
    cen = centers.astype(jnp.int32)
    ctx = contexts.astype(jnp.int32)
    neg = neg_samples.astype(jnp.int32).reshape(-1)
    embT = jnp.transpose(emb)
    table = _transpose(embT, embT)
    scores = _sc_scores(table, _pack_row(cen), _pack_row(ctx),
                        _pack_row(neg), _pack_off(cen), _pack_off(ctx),
                        _pack_off(neg))
    return _loss(scores)[0, 0]


def kernel(centers, contexts, neg_samples, emb):
    return _impl(centers, contexts, neg_samples, emb)
